# baseline (speedup 1.0000x reference)
import functools

import jax
import jax.numpy as jnp
from jax import lax
from jax.experimental import pallas as pl
from jax.experimental.pallas import tpu as pltpu
from jax.experimental.pallas import tpu_sc as plsc

N_SPARSE = 26
N_DENSE = 13
VOCAB = 100000
DIM = 32
B = 4096
EMB_W = N_SPARSE * DIM
OUT_W = EMB_W + N_DENSE

NC = 2
NS = 16
NW = NC * NS
BPW = B // NW
RPB = 16
NBLK = BPW // RPB


def _make_sc_embed():
    mesh = plsc.VectorSubcoreMesh(core_axis_name="c", subcore_axis_name="s")

    @functools.partial(
        pl.kernel,
        mesh=mesh,
        out_type=jax.ShapeDtypeStruct((B, OUT_W), jnp.float32),
        scratch_types=[
            pltpu.VMEM((RPB * EMB_W,), jnp.int32),
            pltpu.VMEM((BPW, EMB_W), jnp.float32),
            pltpu.VMEM((BPW, N_DENSE), jnp.float32),
            pltpu.SemaphoreType.DMA,
        ],
        compiler_params=pltpu.CompilerParams(use_tc_tiling_on_sc=False),
    )
    def sc_embed(dense_hbm, eidx_hbm, tables_hbm, out_hbm,
                 idx_v, asm_v, dense_v, sem):
        wid = lax.axis_index("s") * NC + lax.axis_index("c")
        base = wid * BPW
        pltpu.sync_copy(dense_hbm.at[pl.ds(base, BPW)], dense_v)

        def blk_body(blk, _):
            row0 = base + blk * RPB
            pltpu.sync_copy(eidx_hbm.at[pl.ds(row0 * EMB_W, RPB * EMB_W)],
                            idx_v)
            copies = []
            for rr in range(RPB):
                copies.append(pltpu.async_copy(
                    tables_hbm.at[idx_v.at[pl.ds(rr * EMB_W, EMB_W)]],
                    asm_v.at[blk * RPB + rr],
                    sem))
            for cp in copies:
                cp.wait()
            return 0

        lax.fori_loop(0, NBLK, blk_body, 0)
        pltpu.sync_copy(asm_v, out_hbm.at[pl.ds(base, BPW), pl.ds(0, EMB_W)])
        pltpu.sync_copy(dense_v,
                        out_hbm.at[pl.ds(base, BPW), pl.ds(EMB_W, N_DENSE)])

    return sc_embed


def kernel(x_dense, x_sparse, tables):
    offs = (jnp.arange(N_SPARSE, dtype=jnp.int32) * (DIM * VOCAB))[:, None] \
        + (jnp.arange(DIM, dtype=jnp.int32) * VOCAB)[None, :]
    eidx = (x_sparse[:, :, None] + offs[None, :, :]).reshape(
        B * EMB_W // 128, 128).reshape(-1)
    tables_e = jnp.transpose(tables, (0, 2, 1)).reshape(-1)
    return _make_sc_embed()(x_dense, eidx, tables_e)


# speedup vs baseline: 1.0718x; 1.0718x over previous
import functools

import jax
import jax.numpy as jnp
from jax import lax
from jax.experimental import pallas as pl
from jax.experimental.pallas import tpu as pltpu
from jax.experimental.pallas import tpu_sc as plsc

N_SPARSE = 26
N_DENSE = 13
VOCAB = 100000
DIM = 32
B = 4096
EMB_W = N_SPARSE * DIM
OUT_W = EMB_W + N_DENSE

NC = 2
NS = 16
NW = NC * NS
NU = N_SPARSE * 4
VTAIL = 99968
CHUNK = 10880
CHUNKS = [(i * CHUNK, CHUNK) for i in range(9)] + [(9 * CHUNK, 2048)]
NGRP = B // 16


def _make_sc_embed():
    mesh = plsc.VectorSubcoreMesh(core_axis_name="c", subcore_axis_name="s")

    @functools.partial(
        pl.kernel,
        mesh=mesh,
        out_type=jax.ShapeDtypeStruct((848, B), jnp.float32),
        scratch_types=[
            pltpu.VMEM((8, CHUNK), jnp.float32),
            pltpu.VMEM((B,), jnp.int32),
            pltpu.VMEM((8, B), jnp.float32),
            pltpu.SemaphoreType.DMA,
        ],
        compiler_params=pltpu.CompilerParams(needs_layout_passes=False),
    )
    def sc_embed(xs1d_hbm, xd_hbm, tails_hbm, tables_hbm, out_hbm,
                 chunk_v, xs_v, res_v, sem):
        wid = lax.axis_index("s") * NC + lax.axis_index("c")

        def extract(c0, width, limit):
            def grp(g, _):
                xs16 = xs_v[pl.ds(g * 16, 16)]
                m = (xs16 >= c0) & (xs16 < c0 + limit)
                local = jnp.clip(xs16 - c0, 0, width - 1)
                pos = lax.iota(jnp.int32, 16) + g * 16
                for d in range(8):
                    row = jnp.full((16,), d, jnp.int32)
                    v = plsc.load_gather(chunk_v, [row, local])
                    plsc.store_scatter(res_v, [row, pos], v, mask=m)
                return 0
            lax.fori_loop(0, NGRP, grp, 0)

        def do_unit(u):
            f = u // 4
            tr8 = pl.multiple_of((u % 4) * 8, 8)
            pltpu.sync_copy(xs1d_hbm.at[pl.ds(f * B, B)], xs_v)
            for (c0, width) in CHUNKS:
                pltpu.sync_copy(
                    tables_hbm.at[f, pl.ds(tr8, 8), pl.ds(c0, width)],
                    chunk_v.at[:, pl.ds(0, width)])
                extract(c0, width, width)
            pltpu.sync_copy(tails_hbm.at[f, pl.ds(tr8, 8)],
                            chunk_v.at[:, pl.ds(0, 128)])
            extract(VTAIL, 128, VOCAB - VTAIL)
            pltpu.sync_copy(res_v,
                            out_hbm.at[pl.ds(pl.multiple_of(u * 8, 8), 8)])

        for k in range(4):
            u = wid + NW * k
            if k < 3:
                do_unit(u)
            else:
                @pl.when(u < NU)
                def _():
                    do_unit(u)

        @pl.when(wid == 8)
        def _():
            pltpu.sync_copy(xd_hbm.at[pl.ds(0, 8)], chunk_v.at[:, pl.ds(0, B)])
            pltpu.sync_copy(chunk_v.at[:, pl.ds(0, B)],
                            out_hbm.at[pl.ds(EMB_W, 8)])

        @pl.when(wid == 9)
        def _():
            pltpu.sync_copy(xd_hbm.at[pl.ds(8, 8)], chunk_v.at[:, pl.ds(0, B)])
            pltpu.sync_copy(chunk_v.at[:, pl.ds(0, B)],
                            out_hbm.at[pl.ds(EMB_W + 8, 8)])

    return sc_embed


def kernel(x_dense, x_sparse, tables):
    tables_t = jnp.transpose(tables, (0, 2, 1))
    tails = jnp.pad(tables_t[:, :, VTAIL:],
                    ((0, 0), (0, 0), (0, 128 - (VOCAB - VTAIL))))
    xs1d = jnp.transpose(x_sparse).reshape(N_SPARSE * B)
    xd16 = jnp.pad(jnp.transpose(x_dense), ((0, 3), (0, 0)))
    out_t = _make_sc_embed()(xs1d, xd16, tails, tables_t)
    return jnp.transpose(out_t)[:, :OUT_W]


# speedup vs baseline: 1.9568x; 1.8257x over previous
import functools

import jax
import jax.numpy as jnp
from jax import lax
from jax.experimental import pallas as pl
from jax.experimental.pallas import tpu as pltpu
from jax.experimental.pallas import tpu_sc as plsc

N_SPARSE = 26
N_DENSE = 13
VOCAB = 100000
DIM = 32
B = 4096
EMB_W = N_SPARSE * DIM
OUT_W = EMB_W + N_DENSE

NC = 2
NS = 16
NW = NC * NS
NU = N_SPARSE * 4
VTAIL = 99968
CHUNK = 4608
CHUNKS = [(i * CHUNK, CHUNK) for i in range(21)] + [(21 * CHUNK, 3200)]
EDGES = [c0 for (c0, _) in CHUNKS] + [VTAIL, VOCAB + 96]


def _make_sc_embed():
    mesh = plsc.VectorSubcoreMesh(core_axis_name="c", subcore_axis_name="s")

    @functools.partial(
        pl.kernel,
        mesh=mesh,
        out_type=jax.ShapeDtypeStruct((848, B), jnp.float32),
        scratch_types=[
            pltpu.VMEM((8, CHUNK), jnp.float32),
            pltpu.VMEM((8, CHUNK), jnp.float32),
            pltpu.VMEM((B,), jnp.int32),
            pltpu.VMEM((B,), jnp.int32),
            pltpu.VMEM((128,), jnp.int32),
            pltpu.VMEM((8, B), jnp.float32),
            pltpu.SemaphoreType.DMA,
            pltpu.SemaphoreType.DMA,
        ],
        compiler_params=pltpu.CompilerParams(needs_layout_passes=False),
    )
    def sc_embed(xs1d_hbm, ord1d_hbm, lo1d_hbm, xd_hbm, tails_hbm,
                 tables_hbm, out_hbm,
                 buf_a, buf_b, xs_v, ord_v, lo_v, res_v, sem_a, sem_b):
        wid = lax.axis_index("s") * NC + lax.axis_index("c")
        bufs = (buf_a, buf_b)
        sems = (sem_a, sem_b)

        def extract(buf, c0, width, limit, g_lo, g_hi):
            def grp(g, _):
                xs16 = xs_v[pl.ds(g * 16, 16)]
                m = (xs16 >= c0) & (xs16 < c0 + limit)
                local = jnp.clip(xs16 - c0, 0, width - 1)
                pos = ord_v[pl.ds(g * 16, 16)]
                for d in range(8):
                    row = jnp.full((16,), d, jnp.int32)
                    v = plsc.load_gather(buf, [row, local])
                    plsc.store_scatter(res_v, [row, pos], v, mask=m)
                return 0
            lax.fori_loop(g_lo, g_hi, grp, 0)

        def do_unit(u):
            f = u // 4
            tr8 = pl.multiple_of((u % 4) * 8, 8)
            pltpu.sync_copy(xs1d_hbm.at[pl.ds(f * B, B)], xs_v)
            pltpu.sync_copy(ord1d_hbm.at[pl.ds(f * B, B)], ord_v)
            pltpu.sync_copy(lo1d_hbm.at[pl.ds(f * 128, 128)], lo_v)
            edge_a = lo_v[pl.ds(0, 16)]
            edge_b = lo_v[pl.ds(16, 16)]

            def edge(i):
                return edge_a[i] if i < 16 else edge_b[i - 16]

            def start(ci):
                c0, width = CHUNKS[ci]
                return pltpu.async_copy(
                    tables_hbm.at[f, pl.ds(tr8, 8), pl.ds(c0, width)],
                    bufs[ci % 2].at[:, pl.ds(0, width)],
                    sems[ci % 2])

            cp = start(0)
            for ci, (c0, width) in enumerate(CHUNKS):
                nxt = start(ci + 1) if ci + 1 < len(CHUNKS) else None
                cp.wait()
                g_lo = edge(ci) >> 4
                g_hi = (edge(ci + 1) + 15) >> 4
                extract(bufs[ci % 2], c0, width, width, g_lo, g_hi)
                cp = nxt
            pltpu.sync_copy(tails_hbm.at[f, pl.ds(tr8, 8)],
                            buf_a.at[:, pl.ds(0, 128)])
            g_lo = edge(22) >> 4
            g_hi = (edge(23) + 15) >> 4
            extract(buf_a, VTAIL, 128, VOCAB - VTAIL, g_lo, g_hi)
            pltpu.sync_copy(res_v,
                            out_hbm.at[pl.ds(pl.multiple_of(u * 8, 8), 8)])

        def unit_k(k, _):
            u = wid + NW * k

            @pl.when(u < NU)
            def _():
                do_unit(u)
            return 0

        lax.fori_loop(0, 4, unit_k, 0)

        @pl.when(wid == 8)
        def _():
            pltpu.sync_copy(xd_hbm.at[pl.ds(0, 8)], buf_a.at[:, pl.ds(0, B)])
            pltpu.sync_copy(buf_a.at[:, pl.ds(0, B)],
                            out_hbm.at[pl.ds(EMB_W, 8)])

        @pl.when(wid == 9)
        def _():
            pltpu.sync_copy(xd_hbm.at[pl.ds(8, 8)], buf_a.at[:, pl.ds(0, B)])
            pltpu.sync_copy(buf_a.at[:, pl.ds(0, B)],
                            out_hbm.at[pl.ds(EMB_W + 8, 8)])

    return sc_embed


def kernel(x_dense, x_sparse, tables):
    tables_t = jnp.transpose(tables, (0, 2, 1))
    tails = jnp.pad(tables_t[:, :, VTAIL:],
                    ((0, 0), (0, 0), (0, 128 - (VOCAB - VTAIL))))
    xs_t = jnp.transpose(x_sparse)
    order = jnp.argsort(xs_t, axis=1).astype(jnp.int32)
    xs_sorted = jnp.take_along_axis(xs_t, order, axis=1)
    edges = jnp.array(EDGES, dtype=jnp.int32)
    lo = jax.vmap(lambda r: jnp.searchsorted(r, edges))(
        xs_sorted).astype(jnp.int32)
    lo1d = jnp.pad(lo, ((0, 0), (0, 128 - lo.shape[1]))).reshape(-1)
    xs1d = xs_sorted.reshape(N_SPARSE * B)
    ord1d = order.reshape(N_SPARSE * B)
    xd16 = jnp.pad(jnp.transpose(x_dense), ((0, 3), (0, 0)))
    out_t = _make_sc_embed()(xs1d, ord1d, lo1d, xd16, tails, tables_t)
    return jnp.transpose(out_t)[:, :OUT_W]


# speedup vs baseline: 2.0014x; 1.0228x over previous
import functools

import jax
import jax.numpy as jnp
from jax import lax
from jax.experimental import pallas as pl
from jax.experimental.pallas import tpu as pltpu
from jax.experimental.pallas import tpu_sc as plsc

N_SPARSE = 26
N_DENSE = 13
VOCAB = 100000
DIM = 32
B = 4096
EMB_W = N_SPARSE * DIM
OUT_W = EMB_W + N_DENSE

NC = 2
NS = 16
NW = NC * NS
NU = N_SPARSE * 4
VTAIL = 99968
CHUNK = 4608
CHUNKS = [(i * CHUNK, CHUNK) for i in range(21)] + [(21 * CHUNK, 3200)]
EDGES = [c0 for (c0, _) in CHUNKS] + [VTAIL, VOCAB + 96]


def _make_sc_embed():
    mesh = plsc.VectorSubcoreMesh(core_axis_name="c", subcore_axis_name="s")

    @functools.partial(
        pl.kernel,
        mesh=mesh,
        out_type=jax.ShapeDtypeStruct((848, B), jnp.float32),
        scratch_types=[
            pltpu.VMEM((8, CHUNK), jnp.float32),
            pltpu.VMEM((8, CHUNK), jnp.float32),
            pltpu.VMEM((B,), jnp.int32),
            pltpu.VMEM((128,), jnp.int32),
            pltpu.VMEM((8, B), jnp.float32),
            pltpu.SemaphoreType.DMA,
            pltpu.SemaphoreType.DMA,
        ],
        compiler_params=pltpu.CompilerParams(needs_layout_passes=False),
    )
    def sc_embed(xs1d_hbm, lo1d_hbm, xd_hbm, tails_hbm,
                 tables_hbm, out_hbm,
                 buf_a, buf_b, xs_v, lo_v, res_v, sem_a, sem_b):
        wid = lax.axis_index("s") * NC + lax.axis_index("c")
        bufs = (buf_a, buf_b)
        sems = (sem_a, sem_b)

        def extract(buf, c0, width, limit, g_lo, g_hi):
            def grp(g, _):
                pk16 = xs_v[pl.ds(g * 16, 16)]
                xs16 = lax.shift_right_logical(pk16, 12)
                pos = lax.bitwise_and(pk16, 4095)
                m = (xs16 >= c0) & (xs16 < c0 + limit)
                local = jnp.clip(xs16 - c0, 0, width - 1)
                for d in range(8):
                    row = jnp.full((16,), d, jnp.int32)
                    v = plsc.load_gather(buf, [row, local])
                    plsc.store_scatter(res_v, [row, pos], v, mask=m)
                return 0
            lax.fori_loop(g_lo, g_hi, grp, 0)

        def do_unit(u):
            f = u // 4
            tr8 = pl.multiple_of((u % 4) * 8, 8)
            pltpu.sync_copy(xs1d_hbm.at[pl.ds(f * B, B)], xs_v)
            pltpu.sync_copy(lo1d_hbm.at[pl.ds(f * 128, 128)], lo_v)
            edge_a = lo_v[pl.ds(0, 16)]
            edge_b = lo_v[pl.ds(16, 16)]

            def edge(i):
                return edge_a[i] if i < 16 else edge_b[i - 16]

            def start(ci):
                c0, width = CHUNKS[ci]
                return pltpu.async_copy(
                    tables_hbm.at[f, pl.ds(tr8, 8), pl.ds(c0, width)],
                    bufs[ci % 2].at[:, pl.ds(0, width)],
                    sems[ci % 2])

            cp = start(0)
            for ci, (c0, width) in enumerate(CHUNKS):
                nxt = start(ci + 1) if ci + 1 < len(CHUNKS) else None
                cp.wait()
                g_lo = edge(ci) >> 4
                g_hi = (edge(ci + 1) + 15) >> 4
                extract(bufs[ci % 2], c0, width, width, g_lo, g_hi)
                cp = nxt
            pltpu.sync_copy(tails_hbm.at[f, pl.ds(tr8, 8)],
                            buf_a.at[:, pl.ds(0, 128)])
            g_lo = edge(22) >> 4
            g_hi = (edge(23) + 15) >> 4
            extract(buf_a, VTAIL, 128, VOCAB - VTAIL, g_lo, g_hi)
            pltpu.sync_copy(res_v,
                            out_hbm.at[pl.ds(pl.multiple_of(u * 8, 8), 8)])

        def unit_k(k, _):
            u = wid + NW * k

            @pl.when(u < NU)
            def _():
                do_unit(u)
            return 0

        lax.fori_loop(0, 4, unit_k, 0)

        @pl.when(wid == 8)
        def _():
            pltpu.sync_copy(xd_hbm.at[pl.ds(0, 8)], buf_a.at[:, pl.ds(0, B)])
            pltpu.sync_copy(buf_a.at[:, pl.ds(0, B)],
                            out_hbm.at[pl.ds(EMB_W, 8)])

        @pl.when(wid == 9)
        def _():
            pltpu.sync_copy(xd_hbm.at[pl.ds(8, 8)], buf_a.at[:, pl.ds(0, B)])
            pltpu.sync_copy(buf_a.at[:, pl.ds(0, B)],
                            out_hbm.at[pl.ds(EMB_W + 8, 8)])

    return sc_embed


def kernel(x_dense, x_sparse, tables):
    tables_t = jnp.transpose(tables, (0, 2, 1))
    tails = jnp.pad(tables_t[:, :, VTAIL:],
                    ((0, 0), (0, 0), (0, 128 - (VOCAB - VTAIL))))
    xs_t = jnp.transpose(x_sparse)
    packed = jnp.sort((xs_t << 12) | jnp.arange(B, dtype=jnp.int32)[None, :],
                      axis=1)
    edges = jnp.array(EDGES, dtype=jnp.int32) << 12
    lo = jax.vmap(lambda r: jnp.searchsorted(r, edges))(
        packed).astype(jnp.int32)
    lo1d = jnp.pad(lo, ((0, 0), (0, 128 - lo.shape[1]))).reshape(-1)
    xs1d = packed.reshape(N_SPARSE * B)
    xd16 = jnp.pad(jnp.transpose(x_dense), ((0, 3), (0, 0)))
    out_t = _make_sc_embed()(xs1d, lo1d, xd16, tails, tables_t)
    return jnp.transpose(out_t)[:, :OUT_W]


# speedup vs baseline: 2.0525x; 1.0255x over previous
import functools

import jax
import jax.numpy as jnp
from jax import lax
from jax.experimental import pallas as pl
from jax.experimental.pallas import tpu as pltpu
from jax.experimental.pallas import tpu_sc as plsc

N_SPARSE = 26
N_DENSE = 13
VOCAB = 100000
DIM = 32
B = 4096
EMB_W = N_SPARSE * DIM
OUT_W = EMB_W + N_DENSE

NC = 2
NS = 16
NW = NC * NS
NU = N_SPARSE * 4
VTAIL = 99968
CHUNK = 5632
CHUNKS = [(i * CHUNK, CHUNK) for i in range(17)] + [(17 * CHUNK, 4224)]
EDGES = [c0 for (c0, _) in CHUNKS] + [VTAIL, VOCAB + 96]


def _make_sc_embed():
    mesh = plsc.VectorSubcoreMesh(core_axis_name="c", subcore_axis_name="s")

    @functools.partial(
        pl.kernel,
        mesh=mesh,
        out_type=jax.ShapeDtypeStruct((848, B), jnp.float32),
        scratch_types=[
            pltpu.VMEM((8, CHUNK), jnp.float32),
            pltpu.VMEM((8, CHUNK), jnp.float32),
            pltpu.VMEM((B,), jnp.int32),
            pltpu.VMEM((128,), jnp.int32),
            pltpu.VMEM((8, B), jnp.float32),
            pltpu.SemaphoreType.DMA,
            pltpu.SemaphoreType.DMA,
        ],
        compiler_params=pltpu.CompilerParams(needs_layout_passes=False),
    )
    def sc_embed(xs1d_hbm, lo1d_hbm, xd_hbm, tails_hbm,
                 tables_hbm, out_hbm,
                 buf_a, buf_b, xs_v, lo_v, res_v, sem_a, sem_b):
        wid = lax.axis_index("s") * NC + lax.axis_index("c")
        bufs = (buf_a, buf_b)
        sems = (sem_a, sem_b)

        def extract(buf, c0, width, limit, g_lo, g_hi):
            def grp(g, _):
                pk16 = xs_v[pl.ds(g * 16, 16)]
                xs16 = lax.shift_right_logical(pk16, 12)
                pos = lax.bitwise_and(pk16, 4095)
                m = (xs16 >= c0) & (xs16 < c0 + limit)
                local = jnp.clip(xs16 - c0, 0, width - 1)
                for d in range(8):
                    row = jnp.full((16,), d, jnp.int32)
                    v = plsc.load_gather(buf, [row, local])
                    plsc.store_scatter(res_v, [row, pos], v, mask=m)
                return 0
            lax.fori_loop(g_lo, g_hi, grp, 0)

        def do_unit(u):
            f = u // 4
            tr8 = pl.multiple_of((u % 4) * 8, 8)
            pltpu.sync_copy(xs1d_hbm.at[pl.ds(f * B, B)], xs_v)
            pltpu.sync_copy(lo1d_hbm.at[pl.ds(f * 128, 128)], lo_v)
            edge_a = lo_v[pl.ds(0, 16)]
            edge_b = lo_v[pl.ds(16, 16)]

            def edge(i):
                return edge_a[i] if i < 16 else edge_b[i - 16]

            def start(ci):
                c0, width = CHUNKS[ci]
                return pltpu.async_copy(
                    tables_hbm.at[f, pl.ds(tr8, 8), pl.ds(c0, width)],
                    bufs[ci % 2].at[:, pl.ds(0, width)],
                    sems[ci % 2])

            cp = start(0)
            for ci, (c0, width) in enumerate(CHUNKS):
                nxt = start(ci + 1) if ci + 1 < len(CHUNKS) else None
                cp.wait()
                g_lo = edge(ci) >> 4
                g_hi = (edge(ci + 1) + 15) >> 4
                extract(bufs[ci % 2], c0, width, width, g_lo, g_hi)
                cp = nxt
            pltpu.sync_copy(tails_hbm.at[f, pl.ds(tr8, 8)],
                            buf_a.at[:, pl.ds(0, 128)])
            g_lo = edge(22) >> 4
            g_hi = (edge(23) + 15) >> 4
            extract(buf_a, VTAIL, 128, VOCAB - VTAIL, g_lo, g_hi)
            pltpu.sync_copy(res_v,
                            out_hbm.at[pl.ds(pl.multiple_of(u * 8, 8), 8)])

        def unit_k(k, _):
            u = wid + NW * k

            @pl.when(u < NU)
            def _():
                do_unit(u)
            return 0

        lax.fori_loop(0, 4, unit_k, 0)

        @pl.when(wid == 8)
        def _():
            pltpu.sync_copy(xd_hbm.at[pl.ds(0, 8)], buf_a.at[:, pl.ds(0, B)])
            pltpu.sync_copy(buf_a.at[:, pl.ds(0, B)],
                            out_hbm.at[pl.ds(EMB_W, 8)])

        @pl.when(wid == 9)
        def _():
            pltpu.sync_copy(xd_hbm.at[pl.ds(8, 8)], buf_a.at[:, pl.ds(0, B)])
            pltpu.sync_copy(buf_a.at[:, pl.ds(0, B)],
                            out_hbm.at[pl.ds(EMB_W + 8, 8)])

    return sc_embed


def kernel(x_dense, x_sparse, tables):
    tables_t = jnp.transpose(tables, (0, 2, 1))
    tails = jnp.pad(tables_t[:, :, VTAIL:],
                    ((0, 0), (0, 0), (0, 128 - (VOCAB - VTAIL))))
    xs_t = jnp.transpose(x_sparse)
    packed = jnp.sort((xs_t << 12) | jnp.arange(B, dtype=jnp.int32)[None, :],
                      axis=1)
    edges = jnp.array(EDGES, dtype=jnp.int32) << 12
    lo = jax.vmap(lambda r: jnp.searchsorted(r, edges))(
        packed).astype(jnp.int32)
    lo1d = jnp.pad(lo, ((0, 0), (0, 128 - lo.shape[1]))).reshape(-1)
    xs1d = packed.reshape(N_SPARSE * B)
    xd16 = jnp.pad(jnp.transpose(x_dense), ((0, 3), (0, 0)))
    out_t = _make_sc_embed()(xs1d, lo1d, xd16, tails, tables_t)
    return jnp.transpose(out_t)[:, :OUT_W]
